# Initial kernel scaffold; baseline (speedup 1.0000x reference)
import jax
import jax.numpy as jnp
from jax.experimental import pallas as pl


def kernel(boxes, image_meta, feature_map_p2, feature_map_p3, feature_map_p4, feature_map_p5):
    raise NotImplementedError("write your pallas kernel here")


# speedup vs baseline: 12.5816x; 12.5816x over previous
import functools

import jax
import jax.numpy as jnp
from jax import lax
from jax.experimental import pallas as pl
from jax.experimental.pallas import tpu as pltpu
from jax.experimental.pallas import tpu_sc as plsc

_POOL = 7
_PP = _POOL * _POOL
_NBOX = 1000
_NPAD = 1024
_C = 256
_LANES = 16
_NTILES = 32
_BOX_PER_TILE = _NPAD // _NTILES
_SIZES = (256, 128, 64, 32)
_BASES = (0, 256 * 256, 256 * 256 + 128 * 128, 256 * 256 + 128 * 128 + 64 * 64)
_TROWS = 256 * 256 + 128 * 128 + 64 * 64 + 32 * 32


def _plan(boxes, image_meta):
    y1 = boxes[0, :, 0]
    x1 = boxes[0, :, 1]
    y2 = boxes[0, :, 2]
    x2 = boxes[0, :, 3]
    h = y2 - y1
    w = x2 - x1
    image_shape = image_meta[0, 4:7]
    image_area = (image_shape[0] * image_shape[1]).astype(jnp.float32)
    roi_level = jnp.log(jnp.sqrt(h * w) / (224.0 / jnp.sqrt(image_area))) / jnp.log(2.0)
    lvl = jnp.minimum(5, jnp.maximum(2, 4 + jnp.round(roi_level).astype(jnp.int32)))

    frac = jnp.arange(_POOL, dtype=jnp.float32)[None, :] / (_POOL - 1)

    def axis_plan(lo, span, H):
        s = lo[:, None] * (H - 1) + frac * (span[:, None] * (H - 1))
        s = lax.optimization_barrier(s)
        f0 = jnp.floor(s)
        lfrac = s - f0
        i0 = jnp.clip(f0.astype(jnp.int32), 0, H - 1)
        i1 = jnp.clip(i0 + 1, 0, H - 1)
        valid = (s >= 0) & (s <= H - 1)
        return i0, i1, lfrac, valid

    def sel(vals, dtype):
        out = vals[0].astype(dtype)
        for k, v in enumerate(vals[1:], start=1):
            out = jnp.where((lvl == 2 + k)[:, None], v.astype(dtype), out)
        return out

    y0s, y1s, lys, vys, x0s, x1s, lxs, vxs, strides, bases = [], [], [], [], [], [], [], [], [], []
    for H, base in zip(_SIZES, _BASES):
        i0, i1, ly, vy = axis_plan(y1, h, H)
        j0, j1, lx, vx = axis_plan(x1, w, H)
        y0s.append(i0); y1s.append(i1); lys.append(ly); vys.append(vy)
        x0s.append(j0); x1s.append(j1); lxs.append(lx); vxs.append(vx)
        strides.append(jnp.full((_NBOX, 1), H, jnp.int32))
        bases.append(jnp.full((_NBOX, 1), base, jnp.int32))

    y0 = sel(y0s, jnp.int32); y1i = sel(y1s, jnp.int32)
    x0 = sel(x0s, jnp.int32); x1i = sel(x1s, jnp.int32)
    ly = sel(lys, jnp.float32); lx = sel(lxs, jnp.float32)
    vy = sel(vys, jnp.bool_); vx = sel(vxs, jnp.bool_)
    stride = sel(strides, jnp.int32)[:, 0]
    rowbase = sel(bases, jnp.int32)[:, 0]

    rb = rowbase[:, None, None]
    st = stride[:, None, None]
    tl = (rb + y0[:, :, None] * st + x0[:, None, :]).reshape(_NBOX, _PP)
    tr = (rb + y0[:, :, None] * st + x1i[:, None, :]).reshape(_NBOX, _PP)
    bl = (rb + y1i[:, :, None] * st + x0[:, None, :]).reshape(_NBOX, _PP)
    br = (rb + y1i[:, :, None] * st + x1i[:, None, :]).reshape(_NBOX, _PP)
    idx = jnp.stack([jnp.concatenate([tl, tr], 1), jnp.concatenate([bl, br], 1)], 1)
    idx = jnp.concatenate([idx, jnp.zeros((_NPAD - _NBOX, 2, 2 * _PP), jnp.int32)], 0)

    v = (vy[:, :, None] & vx[:, None, :]).reshape(_NBOX, _PP).astype(jnp.float32)
    ly2 = jnp.broadcast_to(ly[:, :, None], (_NBOX, _POOL, _POOL)).reshape(_NBOX, _PP)
    lx2 = jnp.broadcast_to(lx[:, None, :], (_NBOX, _POOL, _POOL)).reshape(_NBOX, _PP)
    wtl = (1.0 - ly2) * (1.0 - lx2) * v
    wtr = (1.0 - ly2) * lx2 * v
    wbl = ly2 * (1.0 - lx2) * v
    wbr = ly2 * lx2 * v
    wts = jnp.concatenate([wtl, wtr, wbl, wbr], 1)
    wts = jnp.concatenate([wts, jnp.zeros((_NPAD - _NBOX, 4 * _PP), jnp.float32)], 0)
    wts = jnp.broadcast_to(wts[:, :, None], (_NPAD, 4 * _PP, _LANES))
    return idx, wts


def _sc_gather_combine(table, idx, wts):
    mesh = plsc.VectorSubcoreMesh(core_axis_name="c", subcore_axis_name="s")

    @functools.partial(
        pl.kernel,
        out_type=jax.ShapeDtypeStruct((_NBOX * _PP, _C), jnp.float32),
        mesh=mesh,
        compiler_params=pltpu.CompilerParams(use_tc_tiling_on_sc=False),
        scratch_types=[
            pltpu.VMEM((2, 2 * _PP), jnp.int32),
            pltpu.VMEM((2 * _PP, _C), jnp.float32),
            pltpu.VMEM((2 * _PP, _C), jnp.float32),
            pltpu.VMEM((4 * _PP, _LANES), jnp.float32),
            pltpu.VMEM((_PP, _C), jnp.float32),
            pltpu.SemaphoreType.DMA,
            pltpu.SemaphoreType.DMA,
        ],
    )
    def k(table_hbm, idx_hbm, wts_hbm, out_hbm, idx_v, rows_a, rows_b, w_v, out_v, sem_a, sem_b):
        ci = lax.axis_index("c")
        si = lax.axis_index("s")
        wid = si * 2 + ci
        base = wid * _BOX_PER_TILE
        nb = jnp.minimum(_NBOX - base, _BOX_PER_TILE)

        def box_body(g, carry):
            n = base + g
            pltpu.sync_copy(idx_hbm.at[n], idx_v)
            cp_a = pltpu.async_copy(table_hbm.at[idx_v.at[0]], rows_a, sem_a)
            cp_b = pltpu.async_copy(table_hbm.at[idx_v.at[1]], rows_b, sem_b)
            pltpu.sync_copy(wts_hbm.at[n], w_v)
            cp_a.wait()
            cp_b.wait()

            def pix(p, carry2):
                w0 = w_v[p]
                w1 = w_v[_PP + p]
                w2 = w_v[2 * _PP + p]
                w3 = w_v[3 * _PP + p]
                for c in range(_C // _LANES):
                    sl = pl.ds(c * _LANES, _LANES)
                    out_v[p, sl] = (
                        w0 * rows_a[p, sl]
                        + w1 * rows_a[_PP + p, sl]
                        + w2 * rows_b[p, sl]
                        + w3 * rows_b[_PP + p, sl]
                    )
                return carry2

            lax.fori_loop(0, _PP, pix, 0)
            pltpu.sync_copy(out_v, out_hbm.at[pl.ds(n * _PP, _PP)])
            return carry

        lax.fori_loop(0, nb, box_body, 0)

    return k(table, idx, wts)


def kernel(boxes, image_meta, feature_map_p2, feature_map_p3, feature_map_p4, feature_map_p5):
    idx, wts = _plan(boxes, image_meta)
    table = jnp.concatenate(
        [
            feature_map_p2.reshape(-1, _C),
            feature_map_p3.reshape(-1, _C),
            feature_map_p4.reshape(-1, _C),
            feature_map_p5.reshape(-1, _C),
        ],
        0,
    )
    out = _sc_gather_combine(table, idx, wts)
    return out.reshape(1, _NBOX, _POOL, _POOL, _C)


# speedup vs baseline: 19.6778x; 1.5640x over previous
import functools

import jax
import jax.numpy as jnp
from jax import lax
from jax.experimental import pallas as pl
from jax.experimental.pallas import tpu as pltpu
from jax.experimental.pallas import tpu_sc as plsc

_POOL = 7
_PP = _POOL * _POOL
_NBOX = 1000
_NPAD = 1024
_C = 256
_LANES = 16
_NTILES = 32
_BPT = _NPAD // _NTILES
_WPB = 216
_SIZES = (256, 128, 64, 32)


def _plan(boxes, image_meta):
    y1 = boxes[0, :, 0]
    x1 = boxes[0, :, 1]
    y2 = boxes[0, :, 2]
    x2 = boxes[0, :, 3]
    h = y2 - y1
    w = x2 - x1
    image_shape = image_meta[0, 4:7]
    image_area = (image_shape[0] * image_shape[1]).astype(jnp.float32)
    roi_level = jnp.log(jnp.sqrt(h * w) / (224.0 / jnp.sqrt(image_area))) / jnp.log(2.0)
    lvl = jnp.minimum(5, jnp.maximum(2, 4 + jnp.round(roi_level).astype(jnp.int32)))

    frac = jnp.arange(_POOL, dtype=jnp.float32)[None, :] / (_POOL - 1)

    def axis_plan(lo, span, H):
        s = lo[:, None] * (H - 1) + frac * (span[:, None] * (H - 1))
        s = lax.optimization_barrier(s)
        f0 = jnp.floor(s)
        lfrac = s - f0
        i0 = jnp.clip(f0.astype(jnp.int32), 0, H - 1)
        i1 = jnp.clip(i0 + 1, 0, H - 1)
        valid = (s >= 0) & (s <= H - 1)
        return i0, i1, lfrac, valid

    def sel(vals, dtype):
        out = vals[0].astype(dtype)
        for k, v in enumerate(vals[1:], start=1):
            out = jnp.where((lvl == 2 + k)[:, None], v.astype(dtype), out)
        return out

    y0s, y1s, lys, vys, x0s, x1s, lxs, vxs, strides = [], [], [], [], [], [], [], [], []
    for H in _SIZES:
        i0, i1, ly, vy = axis_plan(y1, h, H)
        j0, j1, lx, vx = axis_plan(x1, w, H)
        y0s.append(i0); y1s.append(i1); lys.append(ly); vys.append(vy)
        x0s.append(j0); x1s.append(j1); lxs.append(lx); vxs.append(vx)
        strides.append(jnp.full((_NBOX, 1), H, jnp.int32))

    y0 = sel(y0s, jnp.int32); y1i = sel(y1s, jnp.int32)
    x0 = sel(x0s, jnp.int32); x1i = sel(x1s, jnp.int32)
    ly = sel(lys, jnp.float32); lx = sel(lxs, jnp.float32)
    vy = sel(vys, jnp.bool_); vx = sel(vxs, jnp.bool_)
    stride = sel(strides, jnp.int32)[:, 0]

    st = stride[:, None, None]
    tl = (y0[:, :, None] * st + x0[:, None, :]).reshape(_NBOX, _PP)
    tr = (y0[:, :, None] * st + x1i[:, None, :]).reshape(_NBOX, _PP)
    bl = (y1i[:, :, None] * st + x0[:, None, :]).reshape(_NBOX, _PP)
    br = (y1i[:, :, None] * st + x1i[:, None, :]).reshape(_NBOX, _PP)
    idx = jnp.stack([jnp.concatenate([tl, tr], 1), jnp.concatenate([bl, br], 1)], 1)
    idx = jnp.concatenate([idx, jnp.zeros((_NPAD - _NBOX, 2, 2 * _PP), jnp.int32)], 0)

    v = (vy[:, :, None] & vx[:, None, :]).reshape(_NBOX, _PP).astype(jnp.float32)
    ly2 = jnp.broadcast_to(ly[:, :, None], (_NBOX, _POOL, _POOL)).reshape(_NBOX, _PP)
    lx2 = jnp.broadcast_to(lx[:, None, :], (_NBOX, _POOL, _POOL)).reshape(_NBOX, _PP)
    wtl = (1.0 - ly2) * (1.0 - lx2) * v
    wtr = (1.0 - ly2) * lx2 * v
    wbl = ly2 * (1.0 - lx2) * v
    wbr = ly2 * lx2 * v
    wts = jnp.concatenate(
        [wtl, wtr, wbl, wbr, jnp.zeros((_NBOX, _WPB - 4 * _PP), jnp.float32)], 1)
    wts = jnp.concatenate([wts, jnp.zeros((_NPAD - _NBOX, _WPB), jnp.float32)], 0)

    lvls = jnp.concatenate([lvl, jnp.full((_NPAD - _NBOX + 16,), 2, jnp.int32)])
    return (idx.reshape(_NTILES, _BPT, 2, 2 * _PP),
            wts.reshape(_NTILES, _BPT, _WPB), lvls)


def _sc_gather_combine(t2, t3, t4, t5, idx, wts, lvls):
    mesh = plsc.VectorSubcoreMesh(core_axis_name="c", subcore_axis_name="s")

    @functools.partial(
        pl.kernel,
        out_type=jax.ShapeDtypeStruct((_NBOX * _PP, _C), jnp.float32),
        mesh=mesh,
        compiler_params=pltpu.CompilerParams(use_tc_tiling_on_sc=False),
        scratch_types=[
            pltpu.VMEM((_BPT, 2, 2 * _PP), jnp.int32),
            pltpu.VMEM((_BPT, _WPB), jnp.float32),
            pltpu.VMEM((48,), jnp.int32),
            pltpu.VMEM((4 * _PP, _C), jnp.float32),
            pltpu.VMEM((4 * _PP, _C), jnp.float32),
            pltpu.VMEM((_PP, _C), jnp.float32),
            pltpu.SemaphoreType.DMA,
            pltpu.SemaphoreType.DMA,
            pltpu.SemaphoreType.DMA,
            pltpu.SemaphoreType.DMA,
            pltpu.SemaphoreType.DMA,
        ],
    )
    def k(t2_hbm, t3_hbm, t4_hbm, t5_hbm, idx_hbm, wts_hbm, lvl_hbm, out_hbm,
          idx_v, w_v, lv_v, rows0, rows1, out_v, sa0, sb0, sa1, sb1, sem_o):
        ci = lax.axis_index("c")
        si = lax.axis_index("s")
        wid = si * 2 + ci
        base = wid * _BPT
        nb = jnp.minimum(_NBOX - base, _BPT)

        pltpu.sync_copy(idx_hbm.at[wid], idx_v)
        pltpu.sync_copy(wts_hbm.at[wid], w_v)
        pltpu.sync_copy(lvl_hbm.at[pl.ds(base, 48)], lv_v)

        rows = (rows0, rows1)
        sems = ((sa0, sb0), (sa1, sb1))
        half_a = pl.ds(0, 2 * _PP)
        half_b = pl.ds(2 * _PP, 2 * _PP)

        def with_table(g, fn):
            lv = lv_v[pl.ds(g, 16)][0]
            lax.cond(
                lv <= 3,
                lambda: lax.cond(lv == 2, lambda: fn(t2_hbm), lambda: fn(t3_hbm)),
                lambda: lax.cond(lv == 4, lambda: fn(t4_hbm), lambda: fn(t5_hbm)),
            )

        def issue(g, slot):
            buf, (sa, sb) = rows[slot], sems[slot]

            def go(tab):
                pltpu.async_copy(tab.at[idx_v.at[g, 0]], buf.at[half_a], sa)
                pltpu.async_copy(tab.at[idx_v.at[g, 1]], buf.at[half_b], sb)

            with_table(g, go)

        def wait_gather(g, slot):
            buf, (sa, sb) = rows[slot], sems[slot]

            def go(tab):
                pltpu.make_async_copy(tab.at[idx_v.at[g, 0]], buf.at[half_a], sa).wait()
                pltpu.make_async_copy(tab.at[idx_v.at[g, 1]], buf.at[half_b], sb).wait()

            with_table(g, go)

        def store_desc(g):
            n = base + g
            return pltpu.make_async_copy(out_v, out_hbm.at[pl.ds(n * _PP, _PP)], sem_o)

        def compute(g, slot):
            buf = rows[slot]

            def pix(p, carry):
                w0 = w_v[g, pl.ds(p, 16)][0]
                w1 = w_v[g, pl.ds(_PP + p, 16)][0]
                w2 = w_v[g, pl.ds(2 * _PP + p, 16)][0]
                w3 = w_v[g, pl.ds(3 * _PP + p, 16)][0]
                for c in range(_C // _LANES):
                    sl = pl.ds(c * _LANES, _LANES)
                    out_v[p, sl] = (
                        w0 * buf[p, sl]
                        + w1 * buf[_PP + p, sl]
                        + w2 * buf[2 * _PP + p, sl]
                        + w3 * buf[3 * _PP + p, sl]
                    )
                return carry

            lax.fori_loop(0, _PP, pix, 0)

        def step(g, slot):
            @pl.when(g + 1 < nb)
            def _():
                issue(g + 1, 1 - slot)

            @pl.when(g < nb)
            def _():
                wait_gather(g, slot)

                @pl.when(g >= 1)
                def _():
                    store_desc(g - 1).wait()

                compute(g, slot)
                store_desc(g).start()

        issue(0, 0)

        def outer(g2, carry):
            step(2 * g2, 0)
            step(2 * g2 + 1, 1)
            return carry

        lax.fori_loop(0, (nb + 1) // 2, outer, 0)
        store_desc(nb - 1).wait()

    return k(t2, t3, t4, t5, idx, wts, lvls)


def kernel(boxes, image_meta, feature_map_p2, feature_map_p3, feature_map_p4, feature_map_p5):
    idx, wts, lvls = _plan(boxes, image_meta)
    out = _sc_gather_combine(
        feature_map_p2.reshape(-1, _C),
        feature_map_p3.reshape(-1, _C),
        feature_map_p4.reshape(-1, _C),
        feature_map_p5.reshape(-1, _C),
        idx, wts, lvls,
    )
    return out.reshape(1, _NBOX, _POOL, _POOL, _C)
